# baseline (speedup 1.0000x reference)
import jax
import jax.numpy as jnp
from jax.experimental import pallas as pl
from jax.experimental.pallas import tpu as pltpu


def _diag_norm_kernel(x_ref, out_ref):
    i = pl.program_id(0)
    tm = out_ref.shape[0]

    row_sum = jnp.sum(x_ref[...], axis=1, keepdims=True)
    d = jax.lax.rsqrt(row_sum)
    scale = d * d

    out_ref[...] = jnp.zeros_like(out_ref)

    xd = x_ref[:, pl.ds(i * tm, tm)]
    rows = jax.lax.broadcasted_iota(jnp.int32, (tm, tm), 0)
    cols = jax.lax.broadcasted_iota(jnp.int32, (tm, tm), 1)
    vals = jnp.where(rows == cols, (xd + 1.0) * scale, 0.0)
    out_ref[:, pl.ds(i * tm, tm)] = vals


def _pick_tm(n):
    for cand in (256, 128):
        if n % cand == 0 and 16 * cand * n <= 48 * 1024 * 1024:
            return cand
    return None


def kernel(x):
    x = jnp.asarray(x, jnp.float32)
    n = x.shape[0]
    tm = _pick_tm(n)
    if tm is None:
        tm = min(n, 128)

    grid = (pl.cdiv(n, tm),)
    return pl.pallas_call(
        _diag_norm_kernel,
        out_shape=jax.ShapeDtypeStruct((n, n), jnp.float32),
        grid=grid,
        in_specs=[pl.BlockSpec((tm, n), lambda i: (i, 0))],
        out_specs=pl.BlockSpec((tm, n), lambda i: (i, 0)),
        compiler_params=pltpu.CompilerParams(
            dimension_semantics=("parallel",),
            vmem_limit_bytes=60 * 1024 * 1024,
        ),
    )(x)


# speedup vs baseline: 1.0429x; 1.0429x over previous
import jax
import jax.numpy as jnp
from jax.experimental import pallas as pl
from jax.experimental.pallas import tpu as pltpu


def _diag_norm_kernel(x_ref, out_ref):
    i = pl.program_id(0)
    tm = out_ref.shape[0]

    row_sum = jnp.sum(x_ref[...], axis=1, keepdims=True)
    d = jax.lax.rsqrt(row_sum)
    scale = d * d

    out_ref[...] = jnp.zeros_like(out_ref)

    xd = x_ref[:, pl.ds(i * tm, tm)]
    rows = jax.lax.broadcasted_iota(jnp.int32, (tm, tm), 0)
    cols = jax.lax.broadcasted_iota(jnp.int32, (tm, tm), 1)
    vals = jnp.where(rows == cols, (xd + 1.0) * scale, 0.0)
    out_ref[:, pl.ds(i * tm, tm)] = vals


def _pick_tm(n):
    for cand in (512, 384, 256, 128):
        if n % cand == 0 and 16 * cand * n <= 48 * 1024 * 1024:
            return cand
    return None


def kernel(x):
    x = jnp.asarray(x, jnp.float32)
    n = x.shape[0]
    tm = _pick_tm(n)
    if tm is None:
        tm = min(n, 128)

    grid = (pl.cdiv(n, tm),)
    return pl.pallas_call(
        _diag_norm_kernel,
        out_shape=jax.ShapeDtypeStruct((n, n), jnp.float32),
        grid=grid,
        in_specs=[pl.BlockSpec((tm, n), lambda i: (i, 0))],
        out_specs=pl.BlockSpec((tm, n), lambda i: (i, 0)),
        compiler_params=pltpu.CompilerParams(
            dimension_semantics=("parallel",),
            vmem_limit_bytes=60 * 1024 * 1024,
        ),
    )(x)
